# baseline (speedup 1.0000x reference)
import functools

import jax
import jax.numpy as jnp
from jax import lax
from jax.experimental import pallas as pl
from jax.experimental.pallas import tpu as pltpu
from jax.experimental.pallas import tpu_sc as plsc

F = 128
NC, NS = 2, 16
NW = NC * NS
B = 128
NBUF = 2
W = 8


def _elu(x):
    return jnp.where(x > 0, x, jnp.exp(x) - 1.0)


def _segment_sum_sc(xp, src3, dst3, zrows, n, k1):
    K0 = dst3.shape[1]
    rpt = -(-n // (NS * 8)) * 8
    nacc = NS * rpt

    mesh = plsc.VectorSubcoreMesh(core_axis_name="c", subcore_axis_name="s")

    @functools.partial(
        pl.kernel,
        out_type=jax.ShapeDtypeStruct((NC, nacc, F), jnp.float32),
        mesh=mesh,
        scratch_types=[
            pltpu.VMEM((W, B), jnp.int32),
            pltpu.VMEM((W, B), jnp.int32),
            pltpu.VMEM((B, F), jnp.float32),
            pltpu.VMEM((B, F), jnp.float32),
            pltpu.VMEM_SHARED((nacc, F), jnp.float32),
            pltpu.SemaphoreType.DMA,
            pltpu.SemaphoreType.DMA,
        ],
    )
    def seg_kernel(x_hbm, src_hbm, dst_hbm, zero_hbm, out_hbm,
                   sidx, didx, gb0, gb1, acc, sm0, sm1):
        c = lax.axis_index("c")
        s = lax.axis_index("s")
        wid = c * NS + s
        gbufs = (gb0, gb1)
        sems = (sm0, sm1)

        pltpu.sync_copy(zero_hbm, acc.at[pl.ds(s * rpt, rpt)])
        plsc.subcore_barrier()

        def window(w, carry):
            pltpu.sync_copy(src_hbm.at[wid, pl.ds(w * W, W)], sidx)
            pltpu.sync_copy(dst_hbm.at[wid, pl.ds(w * W, W)], didx)

            def group(g, carry2):
                j0 = g * NBUF
                cps = [pltpu.async_copy(x_hbm.at[sidx.at[j0 + b]],
                                        gbufs[b], sems[b])
                       for b in range(NBUF)]
                for b in range(NBUF):
                    cps[b].wait()
                    pltpu.sync_copy(gbufs[b], acc.at[didx.at[j0 + b]],
                                    add=True)
                return carry2

            lax.fori_loop(0, W // NBUF, group, 0)
            return carry

        kk = lax.select(c == 0, K0, k1)
        lax.fori_loop(0, kk // W, window, 0)
        plsc.subcore_barrier()

        pltpu.sync_copy(acc.at[pl.ds(s * rpt, rpt)],
                        out_hbm.at[c, pl.ds(s * rpt, rpt)])

    return seg_kernel(xp, src3, dst3, zrows)


def _row_spec(r):
    return pl.BlockSpec((r, F), lambda i: (i, 0))


def _w_spec():
    return pl.BlockSpec((F, F), lambda i: (0, 0))


def _b_spec():
    return pl.BlockSpec((1, F), lambda i: (0, 0))


def _mlp_face(hv128, w1, b1, w2, b2, n, r):
    def body(hv_ref, w1_ref, b1_ref, w2_ref, b2_ref, out_ref):
        t = jnp.dot(hv_ref[...], w1_ref[...],
                    preferred_element_type=jnp.float32) + b1_ref[...]
        t = _elu(t)
        t = jnp.dot(t, w2_ref[...],
                    preferred_element_type=jnp.float32) + b2_ref[...]
        out_ref[...] = _elu(t)

    return pl.pallas_call(
        body,
        grid=(n // r,),
        in_specs=[_row_spec(r), _w_spec(), _b_spec(), _w_spec(), _b_spec()],
        out_specs=_row_spec(r),
        out_shape=jax.ShapeDtypeStruct((n, F), jnp.float32),
    )(hv128, w1, b1, w2, b2)


def _stage1(ff, parts, wrel, brel, wroot, w1, b1, w2, b2, w3, b3, n, r):
    def body(ff_ref, p_ref, wrel_ref, brel_ref, wroot_ref,
             w1_ref, b1_ref, w2_ref, b2_ref, w3_ref, b3_ref, out_ref):
        agg = p_ref[0] + p_ref[1]
        t = (jnp.dot(agg, wrel_ref[...], preferred_element_type=jnp.float32)
             + brel_ref[...]
             + jnp.dot(ff_ref[...], wroot_ref[...],
                       preferred_element_type=jnp.float32))
        h = _elu(t)
        h = _elu(jnp.dot(h, w1_ref[...],
                         preferred_element_type=jnp.float32) + b1_ref[...])
        h = _elu(jnp.dot(h, w2_ref[...],
                         preferred_element_type=jnp.float32) + b2_ref[...])
        h = _elu(jnp.dot(h, w3_ref[...],
                         preferred_element_type=jnp.float32) + b3_ref[...])
        out_ref[...] = h

    return pl.pallas_call(
        body,
        grid=(n // r,),
        in_specs=[_row_spec(r),
                  pl.BlockSpec((NC, r, F), lambda i: (0, i, 0)),
                  _w_spec(), _b_spec(), _w_spec(),
                  _w_spec(), _b_spec(), _w_spec(), _b_spec(),
                  _w_spec(), _b_spec()],
        out_specs=_row_spec(r),
        out_shape=jax.ShapeDtypeStruct((n, F), jnp.float32),
    )(ff, parts, wrel, brel, wroot, w1, b1, w2, b2, w3, b3)


def _stage2(ff, h, parts, wrel, brel, wroot,
            w1a, w1b, b1, w2, b2, w3p, b3p, n, r):
    def body(ff_ref, h_ref, p_ref, wrel_ref, brel_ref, wroot_ref,
             w1a_ref, w1b_ref, b1_ref, w2_ref, b2_ref, w3_ref, b3_ref,
             out_ref):
        agg = p_ref[0] + p_ref[1]
        s1_out = (jnp.dot(agg, wrel_ref[...],
                          preferred_element_type=jnp.float32)
                  + brel_ref[...]
                  + jnp.dot(h_ref[...], wroot_ref[...],
                            preferred_element_type=jnp.float32))
        o = _elu(jnp.dot(ff_ref[...], w1a_ref[...],
                         preferred_element_type=jnp.float32)
                 + jnp.dot(s1_out, w1b_ref[...],
                           preferred_element_type=jnp.float32)
                 + b1_ref[...])
        o = _elu(jnp.dot(o, w2_ref[...],
                         preferred_element_type=jnp.float32) + b2_ref[...])
        o = jnp.dot(o, w3_ref[...],
                    preferred_element_type=jnp.float32) + b3_ref[...]
        out_ref[...] = o

    return pl.pallas_call(
        body,
        grid=(n // r,),
        in_specs=[_row_spec(r), _row_spec(r),
                  pl.BlockSpec((NC, r, F), lambda i: (0, i, 0)),
                  _w_spec(), _b_spec(), _w_spec(),
                  _w_spec(), _w_spec(), _b_spec(),
                  _w_spec(), _b_spec(), _w_spec(), _b_spec()],
        out_specs=_row_spec(r),
        out_shape=jax.ShapeDtypeStruct((n, F), jnp.float32),
    )(ff, h, parts, wrel, brel, wroot, w1a, w1b, b1, w2, b2, w3p, b3p)


def kernel(hv, adj, mlp_W1, mlp_b1, mlp_W2, mlp_b2,
           g1_Wrel, g1_brel, g1_Wroot,
           s1_W1, s1_b1, s1_W2, s1_b2, s1_W3, s1_b3,
           g2_Wrel, g2_brel, g2_Wroot,
           s2_W1, s2_b1, s2_W2, s2_b2, s2_W3, s2_b3):
    n = hv.shape[0]
    e = adj.shape[1]
    r = 1000

    ktot = 2 * (-(-e // (NW * B)))
    k0 = -(-(-(-ktot * 65 // 100)) // W) * W
    k1 = max(W, -(-(ktot - k0) // W) * W)
    cap = NS * (k0 + k1) * B
    src_f = jnp.concatenate([adj[0], jnp.full((cap - e,), n, jnp.int32)])
    dst_f = jnp.concatenate([adj[1], jnp.zeros((cap - e,), jnp.int32)])
    c0e = NS * k0 * B
    src_p = jnp.concatenate([
        src_f[:c0e].reshape(NS, k0, B),
        jnp.concatenate([src_f[c0e:].reshape(NS, k1, B),
                         jnp.full((NS, k0 - k1, B), n, jnp.int32)],
                        axis=1)])
    dst_p = jnp.concatenate([
        dst_f[:c0e].reshape(NS, k0, B),
        jnp.concatenate([dst_f[c0e:].reshape(NS, k1, B),
                         jnp.zeros((NS, k0 - k1, B), jnp.int32)],
                        axis=1)])
    zrows = jnp.zeros((-(-n // (NS * 8)) * 8, F), jnp.float32)

    hv128 = jnp.pad(hv, ((0, 0), (0, F - hv.shape[1])))
    w1p = jnp.pad(mlp_W1, ((0, F - mlp_W1.shape[0]), (0, 0)))
    w3p = jnp.pad(s2_W3, ((0, 0), (0, F - s2_W3.shape[1])))
    b3p = jnp.pad(s2_b3, ((0, F - s2_b3.shape[0]),)).reshape(1, F)
    s2_W1a, s2_W1b = s2_W1[:F], s2_W1[F:]
    _r1 = lambda v: v.reshape(1, F)

    ff = _mlp_face(hv128, w1p, _r1(mlp_b1), mlp_W2, _r1(mlp_b2), n, r)

    ffp = jnp.pad(ff, ((0, NS), (0, 0)))
    parts1 = _segment_sum_sc(ffp, src_p, dst_p, zrows, n, k1)

    h = _stage1(ff, parts1, g1_Wrel, _r1(g1_brel), g1_Wroot,
                s1_W1, _r1(s1_b1), s1_W2, _r1(s1_b2), s1_W3, _r1(s1_b3),
                n, r)

    hp = jnp.pad(h, ((0, NS), (0, 0)))
    parts2 = _segment_sum_sc(hp, src_p, dst_p, zrows, n, k1)

    o = _stage2(ff, h, parts2, g2_Wrel, _r1(g2_brel), g2_Wroot,
                s2_W1a, s2_W1b, _r1(s2_b1), s2_W2, _r1(s2_b2), w3p, b3p,
                n, r)
    return o[:, :s2_W3.shape[1]]


# speedup vs baseline: 1.6938x; 1.6938x over previous
import functools

import jax
import jax.numpy as jnp
from jax import lax
from jax.experimental import pallas as pl
from jax.experimental.pallas import tpu as pltpu
from jax.experimental.pallas import tpu_sc as plsc

F = 128
NC, NS = 2, 16
NW = NC * NS
B = 128


def _elu(x):
    return jnp.where(x > 0, x, jnp.exp(x) - 1.0)


def _segment_sum_sc(xp, src3, dst3, zrows, n, k1):
    K0 = dst3.shape[1]
    rpt = -(-n // (NS * 8)) * 8
    nacc = NS * rpt

    mesh = plsc.VectorSubcoreMesh(core_axis_name="c", subcore_axis_name="s")

    @functools.partial(
        pl.kernel,
        out_type=jax.ShapeDtypeStruct((NC, nacc, F), jnp.float32),
        mesh=mesh,
        scratch_types=[
            pltpu.VMEM((K0, B), jnp.int32),
            pltpu.VMEM((K0, B), jnp.int32),
            pltpu.VMEM((B, F), jnp.float32),
            pltpu.VMEM_SHARED((nacc, F), jnp.float32),
            pltpu.SemaphoreType.DMA,
        ],
    )
    def seg_kernel(x_hbm, src_hbm, dst_hbm, zero_hbm, out_hbm,
                   sidx, didx, gbuf, acc, sem):
        c = lax.axis_index("c")
        s = lax.axis_index("s")
        wid = c * NS + s

        pltpu.sync_copy(zero_hbm, acc.at[pl.ds(s * rpt, rpt)])
        pltpu.sync_copy(src_hbm.at[wid], sidx)
        pltpu.sync_copy(dst_hbm.at[wid], didx)
        plsc.subcore_barrier()

        def body(j, carry):
            pltpu.async_copy(x_hbm.at[sidx.at[j]], gbuf, sem).wait()
            pltpu.sync_copy(gbuf, acc.at[didx.at[j]], add=True)
            return carry

        kk = lax.select(c == 0, K0, k1)
        lax.fori_loop(0, kk, body, 0)
        plsc.subcore_barrier()

        pltpu.sync_copy(acc.at[pl.ds(s * rpt, rpt)],
                        out_hbm.at[c, pl.ds(s * rpt, rpt)])

    return seg_kernel(xp, src3, dst3, zrows)


def _row_spec(r):
    return pl.BlockSpec((r, F), lambda i: (i, 0))


def _w_spec():
    return pl.BlockSpec((F, F), lambda i: (0, 0))


def _b_spec():
    return pl.BlockSpec((1, F), lambda i: (0, 0))


def _mlp_face(hv128, w1, b1, w2, b2, n, r):
    def body(hv_ref, w1_ref, b1_ref, w2_ref, b2_ref, out_ref):
        t = jnp.dot(hv_ref[...], w1_ref[...],
                    preferred_element_type=jnp.float32) + b1_ref[...]
        t = _elu(t)
        t = jnp.dot(t, w2_ref[...],
                    preferred_element_type=jnp.float32) + b2_ref[...]
        out_ref[...] = _elu(t)

    return pl.pallas_call(
        body,
        grid=(n // r,),
        in_specs=[_row_spec(r), _w_spec(), _b_spec(), _w_spec(), _b_spec()],
        out_specs=_row_spec(r),
        out_shape=jax.ShapeDtypeStruct((n, F), jnp.float32),
    )(hv128, w1, b1, w2, b2)


def _stage1(ff, parts, wrel, brel, wroot, w1, b1, w2, b2, w3, b3, n, r):
    def body(ff_ref, p_ref, wrel_ref, brel_ref, wroot_ref,
             w1_ref, b1_ref, w2_ref, b2_ref, w3_ref, b3_ref, out_ref):
        agg = p_ref[0] + p_ref[1]
        t = (jnp.dot(agg, wrel_ref[...], preferred_element_type=jnp.float32)
             + brel_ref[...]
             + jnp.dot(ff_ref[...], wroot_ref[...],
                       preferred_element_type=jnp.float32))
        h = _elu(t)
        h = _elu(jnp.dot(h, w1_ref[...],
                         preferred_element_type=jnp.float32) + b1_ref[...])
        h = _elu(jnp.dot(h, w2_ref[...],
                         preferred_element_type=jnp.float32) + b2_ref[...])
        h = _elu(jnp.dot(h, w3_ref[...],
                         preferred_element_type=jnp.float32) + b3_ref[...])
        out_ref[...] = h

    return pl.pallas_call(
        body,
        grid=(n // r,),
        in_specs=[_row_spec(r),
                  pl.BlockSpec((NC, r, F), lambda i: (0, i, 0)),
                  _w_spec(), _b_spec(), _w_spec(),
                  _w_spec(), _b_spec(), _w_spec(), _b_spec(),
                  _w_spec(), _b_spec()],
        out_specs=_row_spec(r),
        out_shape=jax.ShapeDtypeStruct((n, F), jnp.float32),
    )(ff, parts, wrel, brel, wroot, w1, b1, w2, b2, w3, b3)


def _stage2(ff, h, parts, wrel, brel, wroot,
            w1a, w1b, b1, w2, b2, w3p, b3p, n, r):
    def body(ff_ref, h_ref, p_ref, wrel_ref, brel_ref, wroot_ref,
             w1a_ref, w1b_ref, b1_ref, w2_ref, b2_ref, w3_ref, b3_ref,
             out_ref):
        agg = p_ref[0] + p_ref[1]
        s1_out = (jnp.dot(agg, wrel_ref[...],
                          preferred_element_type=jnp.float32)
                  + brel_ref[...]
                  + jnp.dot(h_ref[...], wroot_ref[...],
                            preferred_element_type=jnp.float32))
        o = _elu(jnp.dot(ff_ref[...], w1a_ref[...],
                         preferred_element_type=jnp.float32)
                 + jnp.dot(s1_out, w1b_ref[...],
                           preferred_element_type=jnp.float32)
                 + b1_ref[...])
        o = _elu(jnp.dot(o, w2_ref[...],
                         preferred_element_type=jnp.float32) + b2_ref[...])
        o = jnp.dot(o, w3_ref[...],
                    preferred_element_type=jnp.float32) + b3_ref[...]
        out_ref[...] = o

    return pl.pallas_call(
        body,
        grid=(n // r,),
        in_specs=[_row_spec(r), _row_spec(r),
                  pl.BlockSpec((NC, r, F), lambda i: (0, i, 0)),
                  _w_spec(), _b_spec(), _w_spec(),
                  _w_spec(), _w_spec(), _b_spec(),
                  _w_spec(), _b_spec(), _w_spec(), _b_spec()],
        out_specs=_row_spec(r),
        out_shape=jax.ShapeDtypeStruct((n, F), jnp.float32),
    )(ff, h, parts, wrel, brel, wroot, w1a, w1b, b1, w2, b2, w3p, b3p)


def kernel(hv, adj, mlp_W1, mlp_b1, mlp_W2, mlp_b2,
           g1_Wrel, g1_brel, g1_Wroot,
           s1_W1, s1_b1, s1_W2, s1_b2, s1_W3, s1_b3,
           g2_Wrel, g2_brel, g2_Wroot,
           s2_W1, s2_b1, s2_W2, s2_b2, s2_W3, s2_b3):
    n = hv.shape[0]
    e = adj.shape[1]
    r = 1000

    ktot = 2 * (-(-e // (NW * B)))
    k0 = -(-ktot * 65 // 100)
    k1 = ktot - k0
    cap = NS * ktot * B
    src_f = jnp.concatenate([adj[0], jnp.full((cap - e,), n, jnp.int32)])
    dst_f = jnp.concatenate([adj[1], jnp.zeros((cap - e,), jnp.int32)])
    c0e = NS * k0 * B
    src_p = jnp.concatenate([
        src_f[:c0e].reshape(NS, k0, B),
        jnp.concatenate([src_f[c0e:].reshape(NS, k1, B),
                         jnp.full((NS, k0 - k1, B), n, jnp.int32)],
                        axis=1)])
    dst_p = jnp.concatenate([
        dst_f[:c0e].reshape(NS, k0, B),
        jnp.concatenate([dst_f[c0e:].reshape(NS, k1, B),
                         jnp.zeros((NS, k0 - k1, B), jnp.int32)],
                        axis=1)])
    zrows = jnp.zeros((-(-n // (NS * 8)) * 8, F), jnp.float32)

    hv128 = jnp.pad(hv, ((0, 0), (0, F - hv.shape[1])))
    w1p = jnp.pad(mlp_W1, ((0, F - mlp_W1.shape[0]), (0, 0)))
    w3p = jnp.pad(s2_W3, ((0, 0), (0, F - s2_W3.shape[1])))
    b3p = jnp.pad(s2_b3, ((0, F - s2_b3.shape[0]),)).reshape(1, F)
    s2_W1a, s2_W1b = s2_W1[:F], s2_W1[F:]
    _r1 = lambda v: v.reshape(1, F)

    ff = _mlp_face(hv128, w1p, _r1(mlp_b1), mlp_W2, _r1(mlp_b2), n, r)

    ffp = jnp.pad(ff, ((0, NS), (0, 0)))
    parts1 = _segment_sum_sc(ffp, src_p, dst_p, zrows, n, k1)

    h = _stage1(ff, parts1, g1_Wrel, _r1(g1_brel), g1_Wroot,
                s1_W1, _r1(s1_b1), s1_W2, _r1(s1_b2), s1_W3, _r1(s1_b3),
                n, r)

    hp = jnp.pad(h, ((0, NS), (0, 0)))
    parts2 = _segment_sum_sc(hp, src_p, dst_p, zrows, n, k1)

    o = _stage2(ff, h, parts2, g2_Wrel, _r1(g2_brel), g2_Wroot,
                s2_W1a, s2_W1b, _r1(s2_b1), s2_W2, _r1(s2_b2), w3p, b3p,
                n, r)
    return o[:, :s2_W3.shape[1]]


# speedup vs baseline: 1.7286x; 1.0206x over previous
import functools

import jax
import jax.numpy as jnp
from jax import lax
from jax.experimental import pallas as pl
from jax.experimental.pallas import tpu as pltpu
from jax.experimental.pallas import tpu_sc as plsc

F = 128
NC, NS = 2, 16
NW = NC * NS
B = 128


def _elu(x):
    return jnp.where(x > 0, x, jnp.exp(x) - 1.0)


def _segment_sum_sc(xp, src3, dst3, zrows, n, k1):
    K0 = dst3.shape[1]
    rpt = -(-n // (NS * 8)) * 8
    nacc = NS * rpt

    mesh = plsc.VectorSubcoreMesh(core_axis_name="c", subcore_axis_name="s")

    @functools.partial(
        pl.kernel,
        out_type=jax.ShapeDtypeStruct((NC, nacc, F), jnp.float32),
        mesh=mesh,
        scratch_types=[
            pltpu.VMEM((K0, B), jnp.int32),
            pltpu.VMEM((K0, B), jnp.int32),
            pltpu.VMEM((B, F), jnp.float32),
            pltpu.VMEM_SHARED((nacc, F), jnp.float32),
            pltpu.SemaphoreType.DMA,
        ],
    )
    def seg_kernel(x_hbm, src_hbm, dst_hbm, zero_hbm, out_hbm,
                   sidx, didx, gbuf, acc, sem):
        c = lax.axis_index("c")
        s = lax.axis_index("s")
        wid = c * NS + s

        pltpu.sync_copy(zero_hbm, acc.at[pl.ds(s * rpt, rpt)])
        pltpu.sync_copy(src_hbm.at[wid], sidx)
        pltpu.sync_copy(dst_hbm.at[wid], didx)
        plsc.subcore_barrier()

        def body(j, carry):
            pltpu.async_copy(x_hbm.at[sidx.at[j]], gbuf, sem).wait()
            pltpu.sync_copy(gbuf, acc.at[didx.at[j]], add=True)
            return carry

        kk = lax.select(c == 0, K0, k1)
        lax.fori_loop(0, kk, body, 0)
        plsc.subcore_barrier()

        pltpu.sync_copy(acc.at[pl.ds(s * rpt, rpt)],
                        out_hbm.at[c, pl.ds(s * rpt, rpt)])

    return seg_kernel(xp, src3, dst3, zrows)


def _row_spec(r):
    return pl.BlockSpec((r, F), lambda i: (i, 0))


def _w_spec():
    return pl.BlockSpec((F, F), lambda i: (0, 0))


def _b_spec():
    return pl.BlockSpec((1, F), lambda i: (0, 0))


def _mlp_face(hv128, w1, b1, w2, b2, n, r):
    def body(hv_ref, w1_ref, b1_ref, w2_ref, b2_ref, out_ref):
        t = jnp.dot(hv_ref[...], w1_ref[...],
                    preferred_element_type=jnp.float32) + b1_ref[...]
        t = _elu(t)
        t = jnp.dot(t, w2_ref[...],
                    preferred_element_type=jnp.float32) + b2_ref[...]
        out_ref[...] = _elu(t)

    return pl.pallas_call(
        body,
        grid=(n // r,),
        in_specs=[_row_spec(r), _w_spec(), _b_spec(), _w_spec(), _b_spec()],
        out_specs=_row_spec(r),
        out_shape=jax.ShapeDtypeStruct((n, F), jnp.float32),
    )(hv128, w1, b1, w2, b2)


def _stage1(ff, parts, wrel, brel, wroot, w1, b1, w2, b2, w3, b3, n, r):
    def body(ff_ref, p_ref, wrel_ref, brel_ref, wroot_ref,
             w1_ref, b1_ref, w2_ref, b2_ref, w3_ref, b3_ref, out_ref):
        agg = p_ref[0] + p_ref[1]
        t = (jnp.dot(agg, wrel_ref[...], preferred_element_type=jnp.float32)
             + brel_ref[...]
             + jnp.dot(ff_ref[...], wroot_ref[...],
                       preferred_element_type=jnp.float32))
        h = _elu(t)
        h = _elu(jnp.dot(h, w1_ref[...],
                         preferred_element_type=jnp.float32) + b1_ref[...])
        h = _elu(jnp.dot(h, w2_ref[...],
                         preferred_element_type=jnp.float32) + b2_ref[...])
        h = _elu(jnp.dot(h, w3_ref[...],
                         preferred_element_type=jnp.float32) + b3_ref[...])
        out_ref[...] = h

    return pl.pallas_call(
        body,
        grid=(n // r,),
        in_specs=[_row_spec(r),
                  pl.BlockSpec((NC, r, F), lambda i: (0, i, 0)),
                  _w_spec(), _b_spec(), _w_spec(),
                  _w_spec(), _b_spec(), _w_spec(), _b_spec(),
                  _w_spec(), _b_spec()],
        out_specs=_row_spec(r),
        out_shape=jax.ShapeDtypeStruct((n, F), jnp.float32),
    )(ff, parts, wrel, brel, wroot, w1, b1, w2, b2, w3, b3)


def _stage2(ff, h, parts, wrel, brel, wroot,
            w1a, w1b, b1, w2, b2, w3p, b3p, n, r):
    def body(ff_ref, h_ref, p_ref, wrel_ref, brel_ref, wroot_ref,
             w1a_ref, w1b_ref, b1_ref, w2_ref, b2_ref, w3_ref, b3_ref,
             out_ref):
        agg = p_ref[0] + p_ref[1]
        s1_out = (jnp.dot(agg, wrel_ref[...],
                          preferred_element_type=jnp.float32)
                  + brel_ref[...]
                  + jnp.dot(h_ref[...], wroot_ref[...],
                            preferred_element_type=jnp.float32))
        o = _elu(jnp.dot(ff_ref[...], w1a_ref[...],
                         preferred_element_type=jnp.float32)
                 + jnp.dot(s1_out, w1b_ref[...],
                           preferred_element_type=jnp.float32)
                 + b1_ref[...])
        o = _elu(jnp.dot(o, w2_ref[...],
                         preferred_element_type=jnp.float32) + b2_ref[...])
        o = jnp.dot(o, w3_ref[...],
                    preferred_element_type=jnp.float32) + b3_ref[...]
        out_ref[...] = o

    return pl.pallas_call(
        body,
        grid=(n // r,),
        in_specs=[_row_spec(r), _row_spec(r),
                  pl.BlockSpec((NC, r, F), lambda i: (0, i, 0)),
                  _w_spec(), _b_spec(), _w_spec(),
                  _w_spec(), _w_spec(), _b_spec(),
                  _w_spec(), _b_spec(), _w_spec(), _b_spec()],
        out_specs=_row_spec(r),
        out_shape=jax.ShapeDtypeStruct((n, F), jnp.float32),
    )(ff, h, parts, wrel, brel, wroot, w1a, w1b, b1, w2, b2, w3p, b3p)


def kernel(hv, adj, mlp_W1, mlp_b1, mlp_W2, mlp_b2,
           g1_Wrel, g1_brel, g1_Wroot,
           s1_W1, s1_b1, s1_W2, s1_b2, s1_W3, s1_b3,
           g2_Wrel, g2_brel, g2_Wroot,
           s2_W1, s2_b1, s2_W2, s2_b2, s2_W3, s2_b3):
    n = hv.shape[0]
    e = adj.shape[1]
    r = 2000

    ktot = 2 * (-(-e // (NW * B)))
    k0 = -(-ktot * 65 // 100)
    k1 = ktot - k0
    cap = NS * ktot * B
    src_f = jnp.concatenate([adj[0], jnp.full((cap - e,), n, jnp.int32)])
    dst_f = jnp.concatenate([adj[1], jnp.zeros((cap - e,), jnp.int32)])
    c0e = NS * k0 * B
    src_p = jnp.concatenate([
        src_f[:c0e].reshape(NS, k0, B),
        jnp.concatenate([src_f[c0e:].reshape(NS, k1, B),
                         jnp.full((NS, k0 - k1, B), n, jnp.int32)],
                        axis=1)])
    dst_p = jnp.concatenate([
        dst_f[:c0e].reshape(NS, k0, B),
        jnp.concatenate([dst_f[c0e:].reshape(NS, k1, B),
                         jnp.zeros((NS, k0 - k1, B), jnp.int32)],
                        axis=1)])
    zrows = jnp.zeros((-(-n // (NS * 8)) * 8, F), jnp.float32)

    hv128 = jnp.pad(hv, ((0, 0), (0, F - hv.shape[1])))
    w1p = jnp.pad(mlp_W1, ((0, F - mlp_W1.shape[0]), (0, 0)))
    w3p = jnp.pad(s2_W3, ((0, 0), (0, F - s2_W3.shape[1])))
    b3p = jnp.pad(s2_b3, ((0, F - s2_b3.shape[0]),)).reshape(1, F)
    s2_W1a, s2_W1b = s2_W1[:F], s2_W1[F:]
    _r1 = lambda v: v.reshape(1, F)

    ff = _mlp_face(hv128, w1p, _r1(mlp_b1), mlp_W2, _r1(mlp_b2), n, r)

    ffp = jnp.pad(ff, ((0, NS), (0, 0)))
    parts1 = _segment_sum_sc(ffp, src_p, dst_p, zrows, n, k1)

    h = _stage1(ff, parts1, g1_Wrel, _r1(g1_brel), g1_Wroot,
                s1_W1, _r1(s1_b1), s1_W2, _r1(s1_b2), s1_W3, _r1(s1_b3),
                n, r)

    hp = jnp.pad(h, ((0, NS), (0, 0)))
    parts2 = _segment_sum_sc(hp, src_p, dst_p, zrows, n, k1)

    o = _stage2(ff, h, parts2, g2_Wrel, _r1(g2_brel), g2_Wroot,
                s2_W1a, s2_W1b, _r1(s2_b1), s2_W2, _r1(s2_b2), w3p, b3p,
                n, r)
    return o[:, :s2_W3.shape[1]]
